# Initial kernel scaffold; baseline (speedup 1.0000x reference)
import jax
import jax.numpy as jnp
from jax.experimental import pallas as pl


def kernel(edge_index, senders_pos, receivers_pos, edge_dx_, edge_attr, vector_a, vector_b, vector_c, senders_v_t_, senders_w_t_, receivers_v_t_, receivers_w_t_, node_latent, vel, params):
    raise NotImplementedError("write your pallas kernel here")


# speedup vs baseline: 1.1672x; 1.1672x over previous
import functools

import jax
import jax.numpy as jnp
from jax import lax
from jax.experimental import pallas as pl
from jax.experimental.pallas import tpu as pltpu
from jax.experimental.pallas import tpu_sc as plsc

_N = 10000
_E = 320000
_D = 128

_NC = 2
_NS = 16
_L = 16
_NW = _NC * _NS
_EPW = _E // _NW
_GCH = 400
_GNCH = _EPW // _GCH
_SCH = 1000
_SNCH = _EPW // _SCH
_NPT = _N // _NS

_BN = 1000
_BE = 512

_f32 = jnp.float32


def _ln(y, g, b):
    mu = jnp.mean(y, axis=-1, keepdims=True)
    d = y - mu
    var = jnp.mean(d * d, axis=-1, keepdims=True)
    return d * lax.rsqrt(var + 1e-5) * g + b


def _node_pre_body(nl_ref, w1b_ref, wn1_ref, bn1_ref, wn2_ref, bn2_ref,
                   wnw2_ref, bnw2_ref, p_ref, nv_ref, w_ref):
    nl = nl_ref[...]
    p_ref[...] = jnp.dot(nl, w1b_ref[...], preferred_element_type=_f32)
    h = jnp.maximum(jnp.dot(nl, wn1_ref[...], preferred_element_type=_f32)
                    + bn1_ref[...], 0.0)
    nv = jnp.dot(h, wn2_ref[...], preferred_element_type=_f32) + bn2_ref[...]
    nv_ref[...] = nv
    w8 = jnp.dot(h[:, 0:_D], wnw2_ref[...], preferred_element_type=_f32)
    w_ref[...] = w8 + bnw2_ref[...]


def _gather_body(p_hbm, w_hbm, s_hbm, r_hbm, gp_hbm, ws_hbm, wr_hbm,
                 idx_s, idx_r, rows_a, rows_b, ws_v, wr_v,
                 sem_a, sem_b, sem_c, sem_d):
    wid = lax.axis_index("s") * _NC + lax.axis_index("c")

    def chunk(c, carry):
        base = wid * _EPW + c * _GCH
        pltpu.sync_copy(s_hbm.at[pl.ds(base, _GCH)], idx_s)
        pltpu.sync_copy(r_hbm.at[pl.ds(base, _GCH)], idx_r)
        cp_a = pltpu.async_copy(p_hbm.at[idx_s], rows_a, sem_a)
        cp_b = pltpu.async_copy(p_hbm.at[idx_r], rows_b, sem_b)
        cp_c = pltpu.async_copy(w_hbm.at[idx_s], ws_v, sem_c)
        cp_d = pltpu.async_copy(w_hbm.at[idx_r], wr_v, sem_d)
        cp_a.wait()
        cp_b.wait()

        def add_row(rr, c2):
            for j in range(_D // _L):
                sl = pl.ds(j * _L, _L)
                rows_a[rr, sl] = rows_a[rr, sl] + rows_b[rr, sl]
            return c2
        lax.fori_loop(0, _GCH, add_row, 0)
        cp_c.wait()
        cp_d.wait()

        pltpu.sync_copy(rows_a, gp_hbm.at[pl.ds(base, _GCH)])
        pltpu.sync_copy(ws_v, ws_hbm.at[pl.ds(base, _GCH)])
        pltpu.sync_copy(wr_v, wr_hbm.at[pl.ds(base, _GCH)])
        return carry

    lax.fori_loop(0, _GNCH, chunk, 0)


def _edge_body(sm_ref, gp_ref, ws_ref, wr_ref,
               wnf1_ref, bnf1_ref, wnf2_ref, bnf2_ref, gnf_ref, benf_ref,
               wef1_ref, bef1_ref, wef2_ref, bef2_ref, gef_ref, beef_ref,
               wia_ref, wic_ref, bi1_ref, wi2_ref, bi2_ref, gi_ref, bei_ref,
               wd1_ref, bd1_ref, wd2_ref, bd2_ref,
               il_ref, pay_ref):
    sm = sm_ref[...]
    spos = sm[:, 0:3]
    rpos = sm[:, 3:6]
    dx = sm[:, 6:9]
    va = sm[:, 9:12]
    vb = sm[:, 12:15]
    vc = sm[:, 15:18]
    svt = sm[:, 18:21]
    swt = sm[:, 21:24]
    rvt = sm[:, 24:27]
    rwt = sm[:, 27:30]
    attr = sm[:, 30:46]

    def proj(v):
        return jnp.concatenate([
            jnp.sum(va * v, axis=1, keepdims=True),
            jnp.sum(vb * v, axis=1, keepdims=True),
            jnp.sum(vc * v, axis=1, keepdims=True)], axis=1)

    pad2 = jnp.zeros_like(spos[:, 0:2])
    xs = jnp.concatenate([proj(svt), proj(swt), pad2], axis=1)
    xr = jnp.concatenate([-proj(rvt), -proj(rwt), pad2], axis=1)
    nrm = jnp.sqrt(jnp.sum(dx * dx, axis=1, keepdims=True))
    pad15 = jnp.zeros_like(sm[:, 0:15])
    xe = jnp.concatenate([nrm, attr, pad15], axis=1)

    def mlp_ln(x, w1, b1, w2, b2, g, be):
        h = jnp.maximum(jnp.dot(x, w1[...], preferred_element_type=_f32)
                        + b1[...], 0.0)
        y = jnp.dot(h, w2[...], preferred_element_type=_f32) + b2[...]
        return _ln(y, g[...], be[...])

    s_lat = mlp_ln(xs, wnf1_ref, bnf1_ref, wnf2_ref, bnf2_ref, gnf_ref, benf_ref)
    r_lat = mlp_ln(xr, wnf1_ref, bnf1_ref, wnf2_ref, bnf2_ref, gnf_ref, benf_ref)
    e_lat = mlp_ln(xe, wef1_ref, bef1_ref, wef2_ref, bef2_ref, gef_ref, beef_ref)

    x1 = (jnp.dot(s_lat + r_lat, wia_ref[...], preferred_element_type=_f32)
          + gp_ref[...]
          + jnp.dot(e_lat, wic_ref[...], preferred_element_type=_f32)
          + bi1_ref[...])
    h1 = jnp.maximum(x1, 0.0)
    il = _ln(jnp.dot(h1, wi2_ref[...], preferred_element_type=_f32)
             + bi2_ref[...], gi_ref[...], bei_ref[...])
    il_ref[...] = il

    hd = jnp.maximum(jnp.dot(il, wd1_ref[...], preferred_element_type=_f32)
                     + bd1_ref[...], 0.0)
    dec = jnp.dot(hd, wd2_ref[...], preferred_element_type=_f32) + bd2_ref[...]

    fij = dec[:, 0:1] * va + dec[:, 1:2] * vb + dec[:, 2:3] * vc
    aij = dec[:, 3:4] * va + dec[:, 4:5] * vb + dec[:, 5:6] * vc
    dxij = dec[:, 7:8] * va + dec[:, 8:9] * vb + dec[:, 9:10] * vc
    lam = dec[:, 6:7]

    ws = ws_ref[...][:, 0:1]
    wr = wr_ref[...][:, 0:1]
    lever = rpos - (ws * spos + wr * rpos) / (ws + wr)
    t = fij * lam
    crs = jnp.concatenate([
        lever[:, 1:2] * t[:, 2:3] - lever[:, 2:3] * t[:, 1:2],
        lever[:, 2:3] * t[:, 0:1] - lever[:, 0:1] * t[:, 2:3],
        lever[:, 0:1] * t[:, 1:2] - lever[:, 1:2] * t[:, 0:1]], axis=1)
    tau = aij - crs

    ones = jnp.ones_like(lam)
    zpad = jnp.zeros_like(sm[:, 0:6])
    pay_ref[...] = jnp.concatenate([fij, tau, dxij, ones, zpad], axis=1)


def _scatter_body(pay_hbm, r_hbm, out_hbm, pay_v, idx_v, zbuf, accum, sem):
    cid = lax.axis_index("c")
    sid = lax.axis_index("s")
    wid = sid * _NC + cid

    def zrow(rr, c2):
        zbuf[rr, :] = jnp.zeros((_L,), _f32)
        return c2
    lax.fori_loop(0, _NPT, zrow, 0)
    pltpu.sync_copy(zbuf, accum.at[pl.ds(sid * _NPT, _NPT)])
    plsc.subcore_barrier()

    def chunk(c, carry):
        base = wid * _EPW + c * _SCH
        pltpu.sync_copy(r_hbm.at[pl.ds(base, _SCH)], idx_v)
        pltpu.sync_copy(pay_hbm.at[pl.ds(base, _SCH)], pay_v)
        pltpu.sync_copy(pay_v, accum.at[idx_v], add=True)
        return carry
    lax.fori_loop(0, _SNCH, chunk, 0)

    plsc.subcore_barrier()
    pltpu.sync_copy(accum.at[pl.ds(sid * _NPT, _NPT)],
                    out_hbm.at[cid, pl.ds(sid * _NPT, _NPT)])


def _combine_body(part_ref, nv_ref, vel_ref, dv_ref, dw_ref, disp_ref):
    r = part_ref[0] + part_ref[1]
    force = r[:, 0:3]
    torque = r[:, 3:6]
    corr = r[:, 6:9]
    cnt = r[:, 9:10]
    nv = nv_ref[...]
    dv_ref[...] = nv[:, 1:2] * force
    dw_ref[...] = nv[:, 2:3] * torque
    disp_ref[...] = ((vel_ref[...] + nv[:, 3:6]) * nv[:, 6:7]
                     + corr / jnp.maximum(cnt, 1.0))


def kernel(edge_index, senders_pos, receivers_pos, edge_dx_, edge_attr,
           vector_a, vector_b, vector_c, senders_v_t_, senders_w_t_,
           receivers_v_t_, receivers_w_t_, node_latent, vel, params):
    senders = edge_index[0]
    receivers = edge_index[1]

    nf = params['node_feat_encoder']
    ef = params['edge_feat_encoder']
    ie = params['interaction_encoder']
    i1 = params['i1_decoder']
    i2 = params['i2_decoder']
    fs = params['f_scaler']
    nw = params['node_weight_decoder']
    cc = params['compliance_corr_decoder']
    im = params['inv_mass_decoder']
    ii = params['inv_inertia_decoder']
    ed = params['external_dv_decoder']
    vs = params['velocity_scaler']

    def row2(x):
        return x.reshape(1, -1)

    wnf1 = jnp.zeros((8, _D), _f32).at[0:6].set(nf['W1'])
    wef1 = jnp.zeros((32, _D), _f32).at[0:17].set(ef['W1'])
    wia = ie['W1'][0:_D]
    w1b = ie['W1'][_D:2 * _D]
    wic = ie['W1'][2 * _D:3 * _D]

    wd1 = jnp.concatenate([i1['W1'], i2['W1'], fs['W1'], cc['W1']], axis=1)
    bd1 = jnp.concatenate([i1['b1'], i2['b1'], fs['b1'], cc['b1']])
    wd2 = (jnp.zeros((4 * _D, 16), _f32)
           .at[0:_D, 0:3].set(i1['W2'])
           .at[_D:2 * _D, 3:6].set(i2['W2'])
           .at[2 * _D:3 * _D, 6:7].set(fs['W2'])
           .at[3 * _D:4 * _D, 7:10].set(cc['W2']))
    bd2 = (jnp.zeros((16,), _f32)
           .at[0:3].set(i1['b2']).at[3:6].set(i2['b2'])
           .at[6:7].set(fs['b2']).at[7:10].set(cc['b2']))

    wn1 = jnp.concatenate([nw['W1'], im['W1'], ii['W1'], ed['W1'], vs['W1']],
                          axis=1)
    bn1 = jnp.concatenate([nw['b1'], im['b1'], ii['b1'], ed['b1'], vs['b1']])
    wn2 = (jnp.zeros((5 * _D, 16), _f32)
           .at[0:_D, 0:1].set(nw['W2'])
           .at[_D:2 * _D, 1:2].set(im['W2'])
           .at[2 * _D:3 * _D, 2:3].set(ii['W2'])
           .at[3 * _D:4 * _D, 3:6].set(ed['W2'])
           .at[4 * _D:5 * _D, 6:7].set(vs['W2']))
    bn2 = (jnp.zeros((16,), _f32)
           .at[0:1].set(nw['b2']).at[1:2].set(im['b2'])
           .at[2:3].set(ii['b2']).at[3:6].set(ed['b2'])
           .at[6:7].set(vs['b2']))

    full = lambda shape: pl.BlockSpec(shape, lambda i: tuple(0 for _ in shape))
    p_tab, nodevals, w_col = pl.pallas_call(
        _node_pre_body,
        grid=(_N // _BN,),
        in_specs=[pl.BlockSpec((_BN, _D), lambda i: (i, 0)),
                  full((_D, _D)), full((_D, 5 * _D)), full((1, 5 * _D)),
                  full((5 * _D, 16)), full((1, 16)),
                  full((_D, 8)), full((1, 1))],
        out_specs=[pl.BlockSpec((_BN, _D), lambda i: (i, 0)),
                   pl.BlockSpec((_BN, 16), lambda i: (i, 0)),
                   pl.BlockSpec((_BN, 8), lambda i: (i, 0))],
        out_shape=[jax.ShapeDtypeStruct((_N, _D), _f32),
                   jax.ShapeDtypeStruct((_N, 16), _f32),
                   jax.ShapeDtypeStruct((_N, 8), _f32)],
    )(node_latent, w1b, wn1, row2(bn1), wn2, row2(bn2),
      jnp.zeros((_D, 8), _f32).at[:, 0:1].set(nw['W2']), row2(nw['b2']))

    mesh = plsc.VectorSubcoreMesh(core_axis_name="c", subcore_axis_name="s")
    gp, w_s, w_r = pl.kernel(
        _gather_body,
        out_type=(jax.ShapeDtypeStruct((_E, _D), _f32),
                  jax.ShapeDtypeStruct((_E, 8), _f32),
                  jax.ShapeDtypeStruct((_E, 8), _f32)),
        mesh=mesh,
        scratch_types=[
            pltpu.VMEM((_GCH,), jnp.int32),
            pltpu.VMEM((_GCH,), jnp.int32),
            pltpu.VMEM((_GCH, _D), _f32),
            pltpu.VMEM((_GCH, _D), _f32),
            pltpu.VMEM((_GCH, 8), _f32),
            pltpu.VMEM((_GCH, 8), _f32),
            pltpu.SemaphoreType.DMA,
            pltpu.SemaphoreType.DMA,
            pltpu.SemaphoreType.DMA,
            pltpu.SemaphoreType.DMA,
        ],
        compiler_params=pltpu.CompilerParams(use_tc_tiling_on_sc=False),
    )(p_tab, w_col, senders, receivers)

    small = jnp.concatenate(
        [senders_pos, receivers_pos, edge_dx_, vector_a, vector_b, vector_c,
         senders_v_t_, senders_w_t_, receivers_v_t_, receivers_w_t_,
         edge_attr], axis=1)
    il, payload = pl.pallas_call(
        _edge_body,
        grid=(_E // _BE,),
        in_specs=[pl.BlockSpec((_BE, 46), lambda i: (i, 0)),
                  pl.BlockSpec((_BE, _D), lambda i: (i, 0)),
                  pl.BlockSpec((_BE, 8), lambda i: (i, 0)),
                  pl.BlockSpec((_BE, 8), lambda i: (i, 0)),
                  full((8, _D)), full((1, _D)), full((_D, _D)), full((1, _D)),
                  full((1, _D)), full((1, _D)),
                  full((32, _D)), full((1, _D)), full((_D, _D)), full((1, _D)),
                  full((1, _D)), full((1, _D)),
                  full((_D, _D)), full((_D, _D)), full((1, _D)),
                  full((_D, _D)), full((1, _D)), full((1, _D)), full((1, _D)),
                  full((_D, 4 * _D)), full((1, 4 * _D)),
                  full((4 * _D, 16)), full((1, 16))],
        out_specs=[pl.BlockSpec((_BE, _D), lambda i: (i, 0)),
                   pl.BlockSpec((_BE, 16), lambda i: (i, 0))],
        out_shape=[jax.ShapeDtypeStruct((_E, _D), _f32),
                   jax.ShapeDtypeStruct((_E, 16), _f32)],
    )(small, gp, w_s, w_r,
      wnf1, row2(nf['b1']), nf['W2'], row2(nf['b2']),
      row2(nf['gamma']), row2(nf['beta']),
      wef1, row2(ef['b1']), ef['W2'], row2(ef['b2']),
      row2(ef['gamma']), row2(ef['beta']),
      wia, wic, row2(ie['b1']), ie['W2'], row2(ie['b2']),
      row2(ie['gamma']), row2(ie['beta']),
      wd1, row2(bd1), wd2, row2(bd2))

    partials = pl.kernel(
        _scatter_body,
        out_type=jax.ShapeDtypeStruct((_NC, _N, 16), _f32),
        mesh=mesh,
        scratch_types=[
            pltpu.VMEM((_SCH, 16), _f32),
            pltpu.VMEM((_SCH,), jnp.int32),
            pltpu.VMEM((_NPT, 16), _f32),
            pltpu.VMEM_SHARED((_N, 16), _f32),
            pltpu.SemaphoreType.DMA,
        ],
        compiler_params=pltpu.CompilerParams(use_tc_tiling_on_sc=False),
    )(payload, receivers)

    dv, dw, disp = pl.pallas_call(
        _combine_body,
        grid=(_N // _BN,),
        in_specs=[pl.BlockSpec((_NC, _BN, 16), lambda i: (0, i, 0)),
                  pl.BlockSpec((_BN, 16), lambda i: (i, 0)),
                  pl.BlockSpec((_BN, 3), lambda i: (i, 0))],
        out_specs=[pl.BlockSpec((_BN, 3), lambda i: (i, 0)),
                   pl.BlockSpec((_BN, 3), lambda i: (i, 0)),
                   pl.BlockSpec((_BN, 3), lambda i: (i, 0))],
        out_shape=[jax.ShapeDtypeStruct((_N, 3), _f32),
                   jax.ShapeDtypeStruct((_N, 3), _f32),
                   jax.ShapeDtypeStruct((_N, 3), _f32)],
    )(partials, nodevals, vel)

    return (dv, dw, disp, il)


# speedup vs baseline: 2.1235x; 1.8192x over previous
import functools

import numpy as np

import jax
import jax.numpy as jnp
from jax import lax
from jax.experimental import pallas as pl
from jax.experimental.pallas import tpu as pltpu
from jax.experimental.pallas import tpu_sc as plsc

_N = 10000
_E = 320000
_D = 128

_NC = 2
_NS = 16
_L = 16
_NW = _NC * _NS
_EPW = _E // _NW
_GCH = 400
_GNCH = _EPW // _GCH
_SCH = 1000
_SNCH = _EPW // _SCH
_NPT = _N // _NS

_BN = 1000
_BE = 512

_f32 = jnp.float32


def _ln(y, g, b):
    mu = jnp.mean(y, axis=-1, keepdims=True)
    d = y - mu
    var = jnp.mean(d * d, axis=-1, keepdims=True)
    return d * lax.rsqrt(var + 1e-5) * g + b


def _node_pre_body(nl_ref, w1b_ref, wn1_ref, bn1_ref, wn2_ref, bn2_ref,
                   wnw2_ref, bnw2_ref, p_ref, nv_ref, w_ref):
    nl = nl_ref[...]
    p_ref[...] = jnp.dot(nl, w1b_ref[...], preferred_element_type=_f32)
    h = jnp.maximum(jnp.dot(nl, wn1_ref[...], preferred_element_type=_f32)
                    + bn1_ref[...], 0.0)
    nv = jnp.dot(h, wn2_ref[...], preferred_element_type=_f32) + bn2_ref[...]
    nv_ref[...] = nv
    w8 = jnp.dot(h[:, 0:_D], wnw2_ref[...], preferred_element_type=_f32)
    w_ref[...] = w8 + bnw2_ref[...]


def _gather_body(p_hbm, w_hbm, s_hbm, r_hbm, gp_hbm, ws_hbm, wr_hbm,
                 idx_s, idx_r, rows_a, rows_b, ws_v, wr_v,
                 sem_a, sem_b, sem_c, sem_d):
    wid = lax.axis_index("s") * _NC + lax.axis_index("c")

    def chunk(c, carry):
        base = wid * _EPW + c * _GCH
        pltpu.sync_copy(s_hbm.at[pl.ds(base, _GCH)], idx_s)
        pltpu.sync_copy(r_hbm.at[pl.ds(base, _GCH)], idx_r)
        cp_a = pltpu.async_copy(p_hbm.at[idx_s], rows_a, sem_a)
        cp_b = pltpu.async_copy(p_hbm.at[idx_r], rows_b, sem_b)
        cp_c = pltpu.async_copy(w_hbm.at[idx_s], ws_v, sem_c)
        cp_d = pltpu.async_copy(w_hbm.at[idx_r], wr_v, sem_d)
        cp_a.wait()
        cp_b.wait()

        def add_row(rr, c2):
            for j in range(_D // _L):
                sl = pl.ds(j * _L, _L)
                rows_a[rr, sl] = rows_a[rr, sl] + rows_b[rr, sl]
            return c2
        lax.fori_loop(0, _GCH, add_row, 0)
        cp_c.wait()
        cp_d.wait()

        pltpu.sync_copy(rows_a, gp_hbm.at[pl.ds(base, _GCH)])
        pltpu.sync_copy(ws_v, ws_hbm.at[pl.ds(base, _GCH)])
        pltpu.sync_copy(wr_v, wr_hbm.at[pl.ds(base, _GCH)])
        return carry

    lax.fori_loop(0, _GNCH, chunk, 0)


def _edge_body(sm_ref, gp_ref, ws_ref, wr_ref,
               pl_ref, pr_ref, wsf_ref, wrf_ref, pn_ref, w1ef_ref, w1e0_ref,
               bnf1_ref, wnf2_ref, bnf2_ref, gnf_ref, benf_ref,
               bef1_ref, wef2_ref, bef2_ref, gef_ref, beef_ref,
               wia_ref, wic_ref, bi1_ref, wi2_ref, bi2_ref, gi_ref, bei_ref,
               wd1_ref, bd1_ref, wd2_ref, bd2_ref,
               qu_ref, qv2_ref, qfp_ref, qlam_ref, psp_ref, prp_ref, ow_ref,
               qpay_ref, qc_ref, one9_ref,
               il_ref, pay_ref):
    def dot(a, b):
        return jnp.dot(a, b, preferred_element_type=_f32)

    sm = sm_ref[...]
    lm = dot(sm, pl_ref[...]) * dot(sm, pr_ref[...])
    hs = jnp.maximum(dot(lm, wsf_ref[...]) + bnf1_ref[...], 0.0)
    s_lat = _ln(dot(hs, wnf2_ref[...]) + bnf2_ref[...],
                gnf_ref[...], benf_ref[...])
    hr = jnp.maximum(dot(lm, wrf_ref[...]) + bnf1_ref[...], 0.0)
    r_lat = _ln(dot(hr, wnf2_ref[...]) + bnf2_ref[...],
                gnf_ref[...], benf_ref[...])
    normw = jnp.sqrt(dot(lm, pn_ref[...]))
    he = jnp.maximum(normw * w1e0_ref[...] + dot(sm, w1ef_ref[...])
                     + bef1_ref[...], 0.0)
    e_lat = _ln(dot(he, wef2_ref[...]) + bef2_ref[...],
                gef_ref[...], beef_ref[...])

    x1 = (dot(s_lat + r_lat, wia_ref[...]) + gp_ref[...]
          + dot(e_lat, wic_ref[...]) + bi1_ref[...])
    il = _ln(dot(jnp.maximum(x1, 0.0), wi2_ref[...]) + bi2_ref[...],
             gi_ref[...], bei_ref[...])
    il_ref[...] = il

    hd = jnp.maximum(dot(il, wd1_ref[...]) + bd1_ref[...], 0.0)
    dec = dot(hd, wd2_ref[...]) + bd2_ref[...]

    w2p = dot(dec, qu_ref[...]) * dot(sm, qv2_ref[...])
    spp = dot(sm, psp_ref[...])
    rpp = dot(sm, prp_ref[...])
    wsw = dot(ws_ref[...], ow_ref[...])
    wrw = dot(wr_ref[...], ow_ref[...])
    leverw = rpp - (wsw * spp + wrw * rpp) / (wsw + wrw)
    tp = dot(w2p, qfp_ref[...]) * dot(dec, qlam_ref[...])
    cp = leverw * tp
    pay_ref[...] = (dot(w2p, qpay_ref[...]) + dot(cp, qc_ref[...])
                    + one9_ref[...])


def _scatter_body(pay_hbm, r_hbm, out_hbm, pay_v, idx_v, zbuf, accum, sem):
    cid = lax.axis_index("c")
    sid = lax.axis_index("s")
    wid = sid * _NC + cid

    def zrow(rr, c2):
        zbuf[rr, :] = jnp.zeros((_L,), _f32)
        return c2
    lax.fori_loop(0, _NPT, zrow, 0)
    pltpu.sync_copy(zbuf, accum.at[pl.ds(sid * _NPT, _NPT)])
    plsc.subcore_barrier()

    def chunk(c, carry):
        base = wid * _EPW + c * _SCH
        pltpu.sync_copy(r_hbm.at[pl.ds(base, _SCH)], idx_v)
        pltpu.sync_copy(pay_hbm.at[pl.ds(base, _SCH)], pay_v)
        pltpu.sync_copy(pay_v, accum.at[idx_v], add=True)
        return carry
    lax.fori_loop(0, _SNCH, chunk, 0)

    plsc.subcore_barrier()
    pltpu.sync_copy(accum.at[pl.ds(sid * _NPT, _NPT)],
                    out_hbm.at[cid, pl.ds(sid * _NPT, _NPT)])


def _combine_body(part_ref, nv_ref, vel_ref, dv_ref, dw_ref, disp_ref):
    r = part_ref[0] + part_ref[1]
    force = r[:, 0:3]
    torque = r[:, 3:6]
    corr = r[:, 6:9]
    cnt = r[:, 9:10]
    nv = nv_ref[...]
    dv_ref[...] = nv[:, 1:2] * force
    dw_ref[...] = nv[:, 2:3] * torque
    disp_ref[...] = ((vel_ref[...] + nv[:, 3:6]) * nv[:, 6:7]
                     + corr / jnp.maximum(cnt, 1.0))


def kernel(edge_index, senders_pos, receivers_pos, edge_dx_, edge_attr,
           vector_a, vector_b, vector_c, senders_v_t_, senders_w_t_,
           receivers_v_t_, receivers_w_t_, node_latent, vel, params):
    senders = edge_index[0]
    receivers = edge_index[1]

    nf = params['node_feat_encoder']
    ef = params['edge_feat_encoder']
    ie = params['interaction_encoder']
    i1 = params['i1_decoder']
    i2 = params['i2_decoder']
    fs = params['f_scaler']
    nw = params['node_weight_decoder']
    cc = params['compliance_corr_decoder']
    im = params['inv_mass_decoder']
    ii = params['inv_inertia_decoder']
    ed = params['external_dv_decoder']
    vs = params['velocity_scaler']

    def row2(x):
        return x.reshape(1, -1)

    npf = np.float32
    plm = np.zeros((46, 48), npf)
    prm = np.zeros((46, 48), npf)
    for g, gbase in enumerate((18, 21, 24, 27)):
        for k, vbase in enumerate((9, 12, 15)):
            for c in range(3):
                plm[vbase + c, g * 9 + k * 3 + c] = 1
                prm[gbase + c, g * 9 + k * 3 + c] = 1
    for c in range(3):
        plm[6 + c, 36 + c] = 1
        prm[6 + c, 36 + c] = 1
    rep6 = jnp.repeat(nf['W1'], 3, axis=0)
    wsf = jnp.concatenate([rep6, jnp.zeros((30, _D), _f32)])
    wrf = jnp.concatenate([jnp.zeros((18, _D), _f32), -rep6,
                           jnp.zeros((12, _D), _f32)])
    pnm = np.zeros((48, _D), npf)
    pnm[36:39, :] = 1.0
    w1ef = jnp.zeros((46, _D), _f32).at[30:46].set(ef['W1'][1:17])
    w1e0 = ef['W1'][0:1, :]

    qum = np.zeros((16, 48), npf)
    qv2m = np.zeros((46, 48), npf)
    for k in range(3):
        for c in range(3):
            qum[k, k * 3 + c] = 1
            qum[3 + k, 9 + k * 3 + c] = 1
            qum[7 + k, 18 + k * 3 + c] = 1
            for g in range(3):
                qv2m[9 + 3 * k + c, g * 9 + 3 * k + c] = 1
    qfpm = np.zeros((48, 48), npf)
    for out_l, m in enumerate((2, 0, 1, 1, 2, 0)):
        for k in range(3):
            qfpm[3 * k + m, out_l] = 1
    qlamm = np.zeros((16, 48), npf)
    qlamm[6, 0:6] = 1
    pspm = np.zeros((46, 48), npf)
    prpm = np.zeros((46, 48), npf)
    for out_l, m in enumerate((1, 2, 0, 2, 0, 1)):
        pspm[0 + m, out_l] = 1
        prpm[3 + m, out_l] = 1
    owm = np.zeros((8, 48), npf)
    owm[0, :] = 1
    qpaym = np.zeros((48, 16), npf)
    for k in range(3):
        for c in range(3):
            qpaym[3 * k + c, c] = 1
            qpaym[9 + 3 * k + c, 3 + c] = 1
            qpaym[18 + 3 * k + c, 6 + c] = 1
    qcm = np.zeros((48, 16), npf)
    for i in range(3):
        qcm[i, 3 + i] = -1
        qcm[3 + i, 3 + i] = 1
    one9m = np.zeros((1, 16), npf)
    one9m[0, 9] = 1

    wia = ie['W1'][0:_D]
    w1b = ie['W1'][_D:2 * _D]
    wic = ie['W1'][2 * _D:3 * _D]

    wd1 = jnp.concatenate([i1['W1'], i2['W1'], fs['W1'], cc['W1']], axis=1)
    bd1 = jnp.concatenate([i1['b1'], i2['b1'], fs['b1'], cc['b1']])
    wd2 = (jnp.zeros((4 * _D, 16), _f32)
           .at[0:_D, 0:3].set(i1['W2'])
           .at[_D:2 * _D, 3:6].set(i2['W2'])
           .at[2 * _D:3 * _D, 6:7].set(fs['W2'])
           .at[3 * _D:4 * _D, 7:10].set(cc['W2']))
    bd2 = (jnp.zeros((16,), _f32)
           .at[0:3].set(i1['b2']).at[3:6].set(i2['b2'])
           .at[6:7].set(fs['b2']).at[7:10].set(cc['b2']))

    wn1 = jnp.concatenate([nw['W1'], im['W1'], ii['W1'], ed['W1'], vs['W1']],
                          axis=1)
    bn1 = jnp.concatenate([nw['b1'], im['b1'], ii['b1'], ed['b1'], vs['b1']])
    wn2 = (jnp.zeros((5 * _D, 16), _f32)
           .at[0:_D, 0:1].set(nw['W2'])
           .at[_D:2 * _D, 1:2].set(im['W2'])
           .at[2 * _D:3 * _D, 2:3].set(ii['W2'])
           .at[3 * _D:4 * _D, 3:6].set(ed['W2'])
           .at[4 * _D:5 * _D, 6:7].set(vs['W2']))
    bn2 = (jnp.zeros((16,), _f32)
           .at[0:1].set(nw['b2']).at[1:2].set(im['b2'])
           .at[2:3].set(ii['b2']).at[3:6].set(ed['b2'])
           .at[6:7].set(vs['b2']))

    full = lambda shape: pl.BlockSpec(shape, lambda i: tuple(0 for _ in shape))
    p_tab, nodevals, w_col = pl.pallas_call(
        _node_pre_body,
        grid=(_N // _BN,),
        in_specs=[pl.BlockSpec((_BN, _D), lambda i: (i, 0)),
                  full((_D, _D)), full((_D, 5 * _D)), full((1, 5 * _D)),
                  full((5 * _D, 16)), full((1, 16)),
                  full((_D, 8)), full((1, 1))],
        out_specs=[pl.BlockSpec((_BN, _D), lambda i: (i, 0)),
                   pl.BlockSpec((_BN, 16), lambda i: (i, 0)),
                   pl.BlockSpec((_BN, 8), lambda i: (i, 0))],
        out_shape=[jax.ShapeDtypeStruct((_N, _D), _f32),
                   jax.ShapeDtypeStruct((_N, 16), _f32),
                   jax.ShapeDtypeStruct((_N, 8), _f32)],
    )(node_latent, w1b, wn1, row2(bn1), wn2, row2(bn2),
      jnp.zeros((_D, 8), _f32).at[:, 0:1].set(nw['W2']), row2(nw['b2']))

    mesh = plsc.VectorSubcoreMesh(core_axis_name="c", subcore_axis_name="s")
    gp, w_s, w_r = pl.kernel(
        _gather_body,
        out_type=(jax.ShapeDtypeStruct((_E, _D), _f32),
                  jax.ShapeDtypeStruct((_E, 8), _f32),
                  jax.ShapeDtypeStruct((_E, 8), _f32)),
        mesh=mesh,
        scratch_types=[
            pltpu.VMEM((_GCH,), jnp.int32),
            pltpu.VMEM((_GCH,), jnp.int32),
            pltpu.VMEM((_GCH, _D), _f32),
            pltpu.VMEM((_GCH, _D), _f32),
            pltpu.VMEM((_GCH, 8), _f32),
            pltpu.VMEM((_GCH, 8), _f32),
            pltpu.SemaphoreType.DMA,
            pltpu.SemaphoreType.DMA,
            pltpu.SemaphoreType.DMA,
            pltpu.SemaphoreType.DMA,
        ],
        compiler_params=pltpu.CompilerParams(use_tc_tiling_on_sc=False),
    )(p_tab, w_col, senders, receivers)

    small = jnp.concatenate(
        [senders_pos, receivers_pos, edge_dx_, vector_a, vector_b, vector_c,
         senders_v_t_, senders_w_t_, receivers_v_t_, receivers_w_t_,
         edge_attr], axis=1)
    il, payload = pl.pallas_call(
        _edge_body,
        grid=(_E // _BE,),
        in_specs=[pl.BlockSpec((_BE, 46), lambda i: (i, 0)),
                  pl.BlockSpec((_BE, _D), lambda i: (i, 0)),
                  pl.BlockSpec((_BE, 8), lambda i: (i, 0)),
                  pl.BlockSpec((_BE, 8), lambda i: (i, 0)),
                  full((46, 48)), full((46, 48)), full((48, _D)),
                  full((48, _D)), full((48, _D)), full((46, _D)),
                  full((1, _D)),
                  full((1, _D)), full((_D, _D)), full((1, _D)),
                  full((1, _D)), full((1, _D)),
                  full((1, _D)), full((_D, _D)), full((1, _D)),
                  full((1, _D)), full((1, _D)),
                  full((_D, _D)), full((_D, _D)), full((1, _D)),
                  full((_D, _D)), full((1, _D)), full((1, _D)), full((1, _D)),
                  full((_D, 4 * _D)), full((1, 4 * _D)),
                  full((4 * _D, 16)), full((1, 16)),
                  full((16, 48)), full((46, 48)), full((48, 48)),
                  full((16, 48)), full((46, 48)), full((46, 48)),
                  full((8, 48)), full((48, 16)), full((48, 16)),
                  full((1, 16))],
        out_specs=[pl.BlockSpec((_BE, _D), lambda i: (i, 0)),
                   pl.BlockSpec((_BE, 16), lambda i: (i, 0))],
        out_shape=[jax.ShapeDtypeStruct((_E, _D), _f32),
                   jax.ShapeDtypeStruct((_E, 16), _f32)],
    )(small, gp, w_s, w_r,
      jnp.asarray(plm), jnp.asarray(prm), wsf, wrf, jnp.asarray(pnm),
      w1ef, w1e0,
      row2(nf['b1']), nf['W2'], row2(nf['b2']),
      row2(nf['gamma']), row2(nf['beta']),
      row2(ef['b1']), ef['W2'], row2(ef['b2']),
      row2(ef['gamma']), row2(ef['beta']),
      wia, wic, row2(ie['b1']), ie['W2'], row2(ie['b2']),
      row2(ie['gamma']), row2(ie['beta']),
      wd1, row2(bd1), wd2, row2(bd2),
      jnp.asarray(qum), jnp.asarray(qv2m), jnp.asarray(qfpm),
      jnp.asarray(qlamm), jnp.asarray(pspm), jnp.asarray(prpm),
      jnp.asarray(owm), jnp.asarray(qpaym), jnp.asarray(qcm),
      jnp.asarray(one9m))

    partials = pl.kernel(
        _scatter_body,
        out_type=jax.ShapeDtypeStruct((_NC, _N, 16), _f32),
        mesh=mesh,
        scratch_types=[
            pltpu.VMEM((_SCH, 16), _f32),
            pltpu.VMEM((_SCH,), jnp.int32),
            pltpu.VMEM((_NPT, 16), _f32),
            pltpu.VMEM_SHARED((_N, 16), _f32),
            pltpu.SemaphoreType.DMA,
        ],
        compiler_params=pltpu.CompilerParams(use_tc_tiling_on_sc=False),
    )(payload, receivers)

    dv, dw, disp = pl.pallas_call(
        _combine_body,
        grid=(_N // _BN,),
        in_specs=[pl.BlockSpec((_NC, _BN, 16), lambda i: (0, i, 0)),
                  pl.BlockSpec((_BN, 16), lambda i: (i, 0)),
                  pl.BlockSpec((_BN, 3), lambda i: (i, 0))],
        out_specs=[pl.BlockSpec((_BN, 3), lambda i: (i, 0)),
                   pl.BlockSpec((_BN, 3), lambda i: (i, 0)),
                   pl.BlockSpec((_BN, 3), lambda i: (i, 0))],
        out_shape=[jax.ShapeDtypeStruct((_N, 3), _f32),
                   jax.ShapeDtypeStruct((_N, 3), _f32),
                   jax.ShapeDtypeStruct((_N, 3), _f32)],
    )(partials, nodevals, vel)

    return (dv, dw, disp, il)


# speedup vs baseline: 2.7175x; 1.2797x over previous
import functools

import numpy as np

import jax
import jax.numpy as jnp
from jax import lax
from jax.experimental import pallas as pl
from jax.experimental.pallas import tpu as pltpu
from jax.experimental.pallas import tpu_sc as plsc

_N = 10000
_E = 320000
_D = 128

_NC = 2
_NS = 16
_L = 16
_NW = _NC * _NS
_EPW = _E // _NW
_GCH = 400
_GNCH = _EPW // _GCH
_SCH = 1000
_SNCH = _EPW // _SCH
_NPT = _N // _NS

_BN = 1000
_BE = 2000

_f32 = jnp.float32


def _ln(y, g, b):
    mu = jnp.mean(y, axis=-1, keepdims=True)
    d = y - mu
    var = jnp.mean(d * d, axis=-1, keepdims=True)
    return d * lax.rsqrt(var + 1e-5) * g + b


def _node_pre_body(nl_ref, w1b_ref, wn1_ref, bn1_ref, wn2_ref, bn2_ref,
                   wnw2_ref, bnw2_ref, p_ref, nv_ref, w_ref):
    nl = nl_ref[...]
    p_ref[...] = jnp.dot(nl, w1b_ref[...], preferred_element_type=_f32)
    h = jnp.maximum(jnp.dot(nl, wn1_ref[...], preferred_element_type=_f32)
                    + bn1_ref[...], 0.0)
    nv = jnp.dot(h, wn2_ref[...], preferred_element_type=_f32) + bn2_ref[...]
    nv_ref[...] = nv
    w8 = jnp.dot(h[:, 0:_D], wnw2_ref[...], preferred_element_type=_f32)
    w_ref[...] = w8 + bnw2_ref[...]


def _gather_body(p_hbm, w_hbm, s_hbm, r_hbm, gp_hbm, ws_hbm, wr_hbm,
                 idx_s, idx_r, rows_a, rows_b, ws_v, wr_v,
                 sem_a, sem_b, sem_c, sem_d):
    wid = lax.axis_index("s") * _NC + lax.axis_index("c")

    def chunk(c, carry):
        base = wid * _EPW + c * _GCH
        pltpu.sync_copy(s_hbm.at[pl.ds(base, _GCH)], idx_s)
        pltpu.sync_copy(r_hbm.at[pl.ds(base, _GCH)], idx_r)
        cp_a = pltpu.async_copy(p_hbm.at[idx_s], rows_a, sem_a)
        cp_b = pltpu.async_copy(p_hbm.at[idx_r], rows_b, sem_b)
        cp_c = pltpu.async_copy(w_hbm.at[idx_s], ws_v, sem_c)
        cp_d = pltpu.async_copy(w_hbm.at[idx_r], wr_v, sem_d)
        cp_a.wait()
        cp_b.wait()

        def add_row(rr, c2):
            for j in range(_D // _L):
                sl = pl.ds(j * _L, _L)
                rows_a[rr, sl] = rows_a[rr, sl] + rows_b[rr, sl]
            return c2
        lax.fori_loop(0, _GCH, add_row, 0)
        cp_c.wait()
        cp_d.wait()

        pltpu.sync_copy(rows_a, gp_hbm.at[pl.ds(base, _GCH)])
        pltpu.sync_copy(ws_v, ws_hbm.at[pl.ds(base, _GCH)])
        pltpu.sync_copy(wr_v, wr_hbm.at[pl.ds(base, _GCH)])
        return carry

    lax.fori_loop(0, _GNCH, chunk, 0)


def _edge_body(sm_ref, gp_ref, ws_ref, wr_ref,
               pl_ref, pr_ref, wsf_ref, wrf_ref, pn_ref, w1ef_ref, w1e0_ref,
               bnf1_ref, wnf2_ref, bnf2_ref, gnf_ref, benf_ref,
               bef1_ref, wef2_ref, bef2_ref, gef_ref, beef_ref,
               wia_ref, wic_ref, bi1_ref, wi2_ref, bi2_ref, gi_ref, bei_ref,
               wd1_ref, bd1_ref, wd2_ref, bd2_ref,
               qu_ref, qv2_ref, qfp_ref, qlam_ref, psp_ref, prp_ref, ow_ref,
               qpay_ref, qc_ref, one9_ref,
               il_ref, pay_ref):
    def dot(a, b):
        return jnp.dot(a, b, preferred_element_type=_f32)

    sm = sm_ref[...]
    lm = dot(sm, pl_ref[...]) * dot(sm, pr_ref[...])
    hs = jnp.maximum(dot(lm, wsf_ref[...]) + bnf1_ref[...], 0.0)
    s_lat = _ln(dot(hs, wnf2_ref[...]) + bnf2_ref[...],
                gnf_ref[...], benf_ref[...])
    hr = jnp.maximum(dot(lm, wrf_ref[...]) + bnf1_ref[...], 0.0)
    r_lat = _ln(dot(hr, wnf2_ref[...]) + bnf2_ref[...],
                gnf_ref[...], benf_ref[...])
    normw = jnp.sqrt(dot(lm, pn_ref[...]))
    he = jnp.maximum(normw * w1e0_ref[...] + dot(sm, w1ef_ref[...])
                     + bef1_ref[...], 0.0)
    e_lat = _ln(dot(he, wef2_ref[...]) + bef2_ref[...],
                gef_ref[...], beef_ref[...])

    x1 = (dot(s_lat + r_lat, wia_ref[...]) + gp_ref[...]
          + dot(e_lat, wic_ref[...]) + bi1_ref[...])
    il = _ln(dot(jnp.maximum(x1, 0.0), wi2_ref[...]) + bi2_ref[...],
             gi_ref[...], bei_ref[...])
    il_ref[...] = il

    hd = jnp.maximum(dot(il, wd1_ref[...]) + bd1_ref[...], 0.0)
    dec = dot(hd, wd2_ref[...]) + bd2_ref[...]

    w2p = dot(dec, qu_ref[...]) * dot(sm, qv2_ref[...])
    spp = dot(sm, psp_ref[...])
    rpp = dot(sm, prp_ref[...])
    wsw = dot(ws_ref[...], ow_ref[...])
    wrw = dot(wr_ref[...], ow_ref[...])
    leverw = rpp - (wsw * spp + wrw * rpp) / (wsw + wrw)
    tp = dot(w2p, qfp_ref[...]) * dot(dec, qlam_ref[...])
    cp = leverw * tp
    pay_ref[...] = (dot(w2p, qpay_ref[...]) + dot(cp, qc_ref[...])
                    + one9_ref[...])


def _scatter_body(pay_hbm, r_hbm, out_hbm, pay_v, idx_v, zbuf, accum, sem):
    cid = lax.axis_index("c")
    sid = lax.axis_index("s")
    wid = sid * _NC + cid

    def zrow(rr, c2):
        zbuf[rr, :] = jnp.zeros((_L,), _f32)
        return c2
    lax.fori_loop(0, _NPT, zrow, 0)
    pltpu.sync_copy(zbuf, accum.at[pl.ds(sid * _NPT, _NPT)])
    plsc.subcore_barrier()

    def chunk(c, carry):
        base = wid * _EPW + c * _SCH
        pltpu.sync_copy(r_hbm.at[pl.ds(base, _SCH)], idx_v)
        pltpu.sync_copy(pay_hbm.at[pl.ds(base, _SCH)], pay_v)
        pltpu.sync_copy(pay_v, accum.at[idx_v], add=True)
        return carry
    lax.fori_loop(0, _SNCH, chunk, 0)

    plsc.subcore_barrier()
    pltpu.sync_copy(accum.at[pl.ds(sid * _NPT, _NPT)],
                    out_hbm.at[cid, pl.ds(sid * _NPT, _NPT)])


def _combine_body(part_ref, nv_ref, vel_ref, dv_ref, dw_ref, disp_ref):
    r = part_ref[0] + part_ref[1]
    force = r[:, 0:3]
    torque = r[:, 3:6]
    corr = r[:, 6:9]
    cnt = r[:, 9:10]
    nv = nv_ref[...]
    dv_ref[...] = nv[:, 1:2] * force
    dw_ref[...] = nv[:, 2:3] * torque
    disp_ref[...] = ((vel_ref[...] + nv[:, 3:6]) * nv[:, 6:7]
                     + corr / jnp.maximum(cnt, 1.0))


def kernel(edge_index, senders_pos, receivers_pos, edge_dx_, edge_attr,
           vector_a, vector_b, vector_c, senders_v_t_, senders_w_t_,
           receivers_v_t_, receivers_w_t_, node_latent, vel, params):
    senders = edge_index[0]
    receivers = edge_index[1]

    nf = params['node_feat_encoder']
    ef = params['edge_feat_encoder']
    ie = params['interaction_encoder']
    i1 = params['i1_decoder']
    i2 = params['i2_decoder']
    fs = params['f_scaler']
    nw = params['node_weight_decoder']
    cc = params['compliance_corr_decoder']
    im = params['inv_mass_decoder']
    ii = params['inv_inertia_decoder']
    ed = params['external_dv_decoder']
    vs = params['velocity_scaler']

    def row2(x):
        return x.reshape(1, -1)

    npf = np.float32
    plm = np.zeros((46, 48), npf)
    prm = np.zeros((46, 48), npf)
    for g, gbase in enumerate((18, 21, 24, 27)):
        for k, vbase in enumerate((9, 12, 15)):
            for c in range(3):
                plm[vbase + c, g * 9 + k * 3 + c] = 1
                prm[gbase + c, g * 9 + k * 3 + c] = 1
    for c in range(3):
        plm[6 + c, 36 + c] = 1
        prm[6 + c, 36 + c] = 1
    rep6 = jnp.repeat(nf['W1'], 3, axis=0)
    wsf = jnp.concatenate([rep6, jnp.zeros((30, _D), _f32)])
    wrf = jnp.concatenate([jnp.zeros((18, _D), _f32), -rep6,
                           jnp.zeros((12, _D), _f32)])
    pnm = np.zeros((48, _D), npf)
    pnm[36:39, :] = 1.0
    w1ef = jnp.zeros((46, _D), _f32).at[30:46].set(ef['W1'][1:17])
    w1e0 = ef['W1'][0:1, :]

    qum = np.zeros((16, 48), npf)
    qv2m = np.zeros((46, 48), npf)
    for k in range(3):
        for c in range(3):
            qum[k, k * 3 + c] = 1
            qum[3 + k, 9 + k * 3 + c] = 1
            qum[7 + k, 18 + k * 3 + c] = 1
            for g in range(3):
                qv2m[9 + 3 * k + c, g * 9 + 3 * k + c] = 1
    qfpm = np.zeros((48, 48), npf)
    for out_l, m in enumerate((2, 0, 1, 1, 2, 0)):
        for k in range(3):
            qfpm[3 * k + m, out_l] = 1
    qlamm = np.zeros((16, 48), npf)
    qlamm[6, 0:6] = 1
    pspm = np.zeros((46, 48), npf)
    prpm = np.zeros((46, 48), npf)
    for out_l, m in enumerate((1, 2, 0, 2, 0, 1)):
        pspm[0 + m, out_l] = 1
        prpm[3 + m, out_l] = 1
    owm = np.zeros((8, 48), npf)
    owm[0, :] = 1
    qpaym = np.zeros((48, 16), npf)
    for k in range(3):
        for c in range(3):
            qpaym[3 * k + c, c] = 1
            qpaym[9 + 3 * k + c, 3 + c] = 1
            qpaym[18 + 3 * k + c, 6 + c] = 1
    qcm = np.zeros((48, 16), npf)
    for i in range(3):
        qcm[i, 3 + i] = -1
        qcm[3 + i, 3 + i] = 1
    one9m = np.zeros((1, 16), npf)
    one9m[0, 9] = 1

    wia = ie['W1'][0:_D]
    w1b = ie['W1'][_D:2 * _D]
    wic = ie['W1'][2 * _D:3 * _D]

    wd1 = jnp.concatenate([i1['W1'], i2['W1'], fs['W1'], cc['W1']], axis=1)
    bd1 = jnp.concatenate([i1['b1'], i2['b1'], fs['b1'], cc['b1']])
    wd2 = (jnp.zeros((4 * _D, 16), _f32)
           .at[0:_D, 0:3].set(i1['W2'])
           .at[_D:2 * _D, 3:6].set(i2['W2'])
           .at[2 * _D:3 * _D, 6:7].set(fs['W2'])
           .at[3 * _D:4 * _D, 7:10].set(cc['W2']))
    bd2 = (jnp.zeros((16,), _f32)
           .at[0:3].set(i1['b2']).at[3:6].set(i2['b2'])
           .at[6:7].set(fs['b2']).at[7:10].set(cc['b2']))

    wn1 = jnp.concatenate([nw['W1'], im['W1'], ii['W1'], ed['W1'], vs['W1']],
                          axis=1)
    bn1 = jnp.concatenate([nw['b1'], im['b1'], ii['b1'], ed['b1'], vs['b1']])
    wn2 = (jnp.zeros((5 * _D, 16), _f32)
           .at[0:_D, 0:1].set(nw['W2'])
           .at[_D:2 * _D, 1:2].set(im['W2'])
           .at[2 * _D:3 * _D, 2:3].set(ii['W2'])
           .at[3 * _D:4 * _D, 3:6].set(ed['W2'])
           .at[4 * _D:5 * _D, 6:7].set(vs['W2']))
    bn2 = (jnp.zeros((16,), _f32)
           .at[0:1].set(nw['b2']).at[1:2].set(im['b2'])
           .at[2:3].set(ii['b2']).at[3:6].set(ed['b2'])
           .at[6:7].set(vs['b2']))

    full = lambda shape: pl.BlockSpec(shape, lambda i: tuple(0 for _ in shape))
    p_tab, nodevals, w_col = pl.pallas_call(
        _node_pre_body,
        grid=(_N // _BN,),
        in_specs=[pl.BlockSpec((_BN, _D), lambda i: (i, 0)),
                  full((_D, _D)), full((_D, 5 * _D)), full((1, 5 * _D)),
                  full((5 * _D, 16)), full((1, 16)),
                  full((_D, 8)), full((1, 1))],
        out_specs=[pl.BlockSpec((_BN, _D), lambda i: (i, 0)),
                   pl.BlockSpec((_BN, 16), lambda i: (i, 0)),
                   pl.BlockSpec((_BN, 8), lambda i: (i, 0))],
        out_shape=[jax.ShapeDtypeStruct((_N, _D), _f32),
                   jax.ShapeDtypeStruct((_N, 16), _f32),
                   jax.ShapeDtypeStruct((_N, 8), _f32)],
    )(node_latent, w1b, wn1, row2(bn1), wn2, row2(bn2),
      jnp.zeros((_D, 8), _f32).at[:, 0:1].set(nw['W2']), row2(nw['b2']))

    mesh = plsc.VectorSubcoreMesh(core_axis_name="c", subcore_axis_name="s")
    gp, w_s, w_r = pl.kernel(
        _gather_body,
        out_type=(jax.ShapeDtypeStruct((_E, _D), _f32),
                  jax.ShapeDtypeStruct((_E, 8), _f32),
                  jax.ShapeDtypeStruct((_E, 8), _f32)),
        mesh=mesh,
        scratch_types=[
            pltpu.VMEM((_GCH,), jnp.int32),
            pltpu.VMEM((_GCH,), jnp.int32),
            pltpu.VMEM((_GCH, _D), _f32),
            pltpu.VMEM((_GCH, _D), _f32),
            pltpu.VMEM((_GCH, 8), _f32),
            pltpu.VMEM((_GCH, 8), _f32),
            pltpu.SemaphoreType.DMA,
            pltpu.SemaphoreType.DMA,
            pltpu.SemaphoreType.DMA,
            pltpu.SemaphoreType.DMA,
        ],
        compiler_params=pltpu.CompilerParams(use_tc_tiling_on_sc=False),
    )(p_tab, w_col, senders, receivers)

    small = jnp.concatenate(
        [senders_pos, receivers_pos, edge_dx_, vector_a, vector_b, vector_c,
         senders_v_t_, senders_w_t_, receivers_v_t_, receivers_w_t_,
         edge_attr], axis=1)
    il, payload = pl.pallas_call(
        _edge_body,
        grid=(_E // _BE,),
        in_specs=[pl.BlockSpec((_BE, 46), lambda i: (i, 0)),
                  pl.BlockSpec((_BE, _D), lambda i: (i, 0)),
                  pl.BlockSpec((_BE, 8), lambda i: (i, 0)),
                  pl.BlockSpec((_BE, 8), lambda i: (i, 0)),
                  full((46, 48)), full((46, 48)), full((48, _D)),
                  full((48, _D)), full((48, _D)), full((46, _D)),
                  full((1, _D)),
                  full((1, _D)), full((_D, _D)), full((1, _D)),
                  full((1, _D)), full((1, _D)),
                  full((1, _D)), full((_D, _D)), full((1, _D)),
                  full((1, _D)), full((1, _D)),
                  full((_D, _D)), full((_D, _D)), full((1, _D)),
                  full((_D, _D)), full((1, _D)), full((1, _D)), full((1, _D)),
                  full((_D, 4 * _D)), full((1, 4 * _D)),
                  full((4 * _D, 16)), full((1, 16)),
                  full((16, 48)), full((46, 48)), full((48, 48)),
                  full((16, 48)), full((46, 48)), full((46, 48)),
                  full((8, 48)), full((48, 16)), full((48, 16)),
                  full((1, 16))],
        out_specs=[pl.BlockSpec((_BE, _D), lambda i: (i, 0)),
                   pl.BlockSpec((_BE, 16), lambda i: (i, 0))],
        out_shape=[jax.ShapeDtypeStruct((_E, _D), _f32),
                   jax.ShapeDtypeStruct((_E, 16), _f32)],
    )(small, gp, w_s, w_r,
      jnp.asarray(plm), jnp.asarray(prm), wsf, wrf, jnp.asarray(pnm),
      w1ef, w1e0,
      row2(nf['b1']), nf['W2'], row2(nf['b2']),
      row2(nf['gamma']), row2(nf['beta']),
      row2(ef['b1']), ef['W2'], row2(ef['b2']),
      row2(ef['gamma']), row2(ef['beta']),
      wia, wic, row2(ie['b1']), ie['W2'], row2(ie['b2']),
      row2(ie['gamma']), row2(ie['beta']),
      wd1, row2(bd1), wd2, row2(bd2),
      jnp.asarray(qum), jnp.asarray(qv2m), jnp.asarray(qfpm),
      jnp.asarray(qlamm), jnp.asarray(pspm), jnp.asarray(prpm),
      jnp.asarray(owm), jnp.asarray(qpaym), jnp.asarray(qcm),
      jnp.asarray(one9m))

    partials = pl.kernel(
        _scatter_body,
        out_type=jax.ShapeDtypeStruct((_NC, _N, 16), _f32),
        mesh=mesh,
        scratch_types=[
            pltpu.VMEM((_SCH, 16), _f32),
            pltpu.VMEM((_SCH,), jnp.int32),
            pltpu.VMEM((_NPT, 16), _f32),
            pltpu.VMEM_SHARED((_N, 16), _f32),
            pltpu.SemaphoreType.DMA,
        ],
        compiler_params=pltpu.CompilerParams(use_tc_tiling_on_sc=False),
    )(payload, receivers)

    dv, dw, disp = pl.pallas_call(
        _combine_body,
        grid=(_N // _BN,),
        in_specs=[pl.BlockSpec((_NC, _BN, 16), lambda i: (0, i, 0)),
                  pl.BlockSpec((_BN, 16), lambda i: (i, 0)),
                  pl.BlockSpec((_BN, 3), lambda i: (i, 0))],
        out_specs=[pl.BlockSpec((_BN, 3), lambda i: (i, 0)),
                   pl.BlockSpec((_BN, 3), lambda i: (i, 0)),
                   pl.BlockSpec((_BN, 3), lambda i: (i, 0))],
        out_shape=[jax.ShapeDtypeStruct((_N, 3), _f32),
                   jax.ShapeDtypeStruct((_N, 3), _f32),
                   jax.ShapeDtypeStruct((_N, 3), _f32)],
    )(partials, nodevals, vel)

    return (dv, dw, disp, il)
